# baseline (speedup 1.0000x reference)
import functools

import jax
import jax.numpy as jnp
from jax import lax
from jax.experimental import pallas as pl
from jax.experimental.pallas import tpu as pltpu
from jax.experimental.pallas import tpu_sc as plsc

_NC = 2
_NS = 16
_NW = _NC * _NS
_L = 16

_R = 128
_C = 8192
_K = 3
_RPW = _R // _NW
_NV = _C // _L
_OPAD = 16


def _bcast_max(x):
  return plsc.cummax(jnp.flip(plsc.cummax(x)))


def _bcast_min_i32(x):
  return -_bcast_max(-x)


def _topk_body(x_hbm, vals_hbm, idxs_hbm, xin, vout, iout):
  w = lax.axis_index("s") * _NC + lax.axis_index("c")
  base = w * _RPW
  pltpu.sync_copy(x_hbm.at[pl.ds(base, _RPW)], xin)

  neg_inf = jnp.full((_L,), -jnp.inf, jnp.float32)
  zeros_i = jnp.zeros((_L,), jnp.int32)
  iota = lax.iota(jnp.int32, _L)

  for r in range(_RPW):
    def body(t, carry):
      m1, m2, m3, i1, i2, i3, iv = carry
      v = xin[r, pl.ds(t * _L, _L)]
      c1 = v > m1
      c2 = v > m2
      c3 = v > m3
      t1 = jnp.minimum(m1, v)
      n1 = jnp.maximum(m1, v)
      t2 = jnp.minimum(m2, t1)
      n2 = jnp.maximum(m2, t1)
      n3 = jnp.maximum(m3, t2)
      j1 = jnp.where(c1, iv, i1)
      j2 = jnp.where(c1, i1, jnp.where(c2, iv, i2))
      j3 = jnp.where(c2, i2, jnp.where(c3, iv, i3))
      return n1, n2, n3, j1, j2, j3, iv + _L

    m1, m2, m3, i1, i2, i3, _ = lax.fori_loop(
        0, _NV, body,
        (neg_inf, neg_inf, neg_inf, zeros_i, zeros_i, zeros_i, iota))

    big = jnp.full((_L,), jnp.int32(2**30))
    vacc = neg_inf
    iacc = zeros_i
    for k in range(_K):
      s = _bcast_max(m1)
      cand = _bcast_min_i32(jnp.where(m1 == s, i1, big))
      vacc = jnp.where(iota == k, s, vacc)
      iacc = jnp.where(iota == k, cand, iacc)
      hit = (m1 == s) & (i1 == cand)
      m1 = jnp.where(hit, m2, m1)
      i1 = jnp.where(hit, i2, i1)
      m2 = jnp.where(hit, m3, m2)
      i2 = jnp.where(hit, i3, i2)
      m3 = jnp.where(hit, neg_inf, m3)
    vout[r, :] = vacc
    iout[r, :] = iacc

  pltpu.sync_copy(vout, vals_hbm.at[pl.ds(base, _RPW)])
  pltpu.sync_copy(iout, idxs_hbm.at[pl.ds(base, _RPW)])


_topk_sc = functools.partial(
    pl.kernel,
    out_type=(jax.ShapeDtypeStruct((_R, _OPAD), jnp.float32),
              jax.ShapeDtypeStruct((_R, _OPAD), jnp.int32)),
    mesh=plsc.VectorSubcoreMesh(
        core_axis_name="c", subcore_axis_name="s",
        num_cores=_NC, num_subcores=_NS),
    compiler_params=pltpu.CompilerParams(needs_layout_passes=False),
    scratch_types=[
        pltpu.VMEM((_RPW, _C), jnp.float32),
        pltpu.VMEM((_RPW, _OPAD), jnp.float32),
        pltpu.VMEM((_RPW, _OPAD), jnp.int32),
    ],
)(_topk_body)


@jax.jit
def kernel(x):
  vals, idxs = _topk_sc(x)
  return vals[:, :_K], idxs[:, :_K]


# speedup vs baseline: 1.0103x; 1.0103x over previous
import functools

import jax
import jax.numpy as jnp
from jax import lax
from jax.experimental import pallas as pl
from jax.experimental.pallas import tpu as pltpu
from jax.experimental.pallas import tpu_sc as plsc

_NC = 2
_NS = 16
_NW = _NC * _NS
_L = 16

_R = 128
_C = 8192
_K = 3
_RPW = _R // _NW
_NV = _C // _L
_OPAD = 16


def _bcast_max(x):
  return plsc.cummax(jnp.flip(plsc.cummax(x)))


def _bcast_min_i32(x):
  return -_bcast_max(-x)


def _topk_body(x_hbm, vals_hbm, idxs_hbm, xin, vout, iout):
  w = lax.axis_index("s") * _NC + lax.axis_index("c")
  base = w * _RPW
  pltpu.sync_copy(x_hbm.at[pl.ds(base, _RPW)], xin)

  neg_inf = jnp.full((_L,), -jnp.inf, jnp.float32)
  zeros_i = jnp.zeros((_L,), jnp.int32)
  iota = lax.iota(jnp.int32, _L)
  big = jnp.full((_L,), jnp.int32(2**30))

  def row_body(r, _):
    def body(t, carry):
      m1, m2, m3, i1, i2, i3, iv = carry
      v = xin[r, pl.ds(t * _L, _L)]
      c1 = v > m1
      c2 = v > m2
      c3 = v > m3
      t1 = jnp.minimum(m1, v)
      n1 = jnp.maximum(m1, v)
      t2 = jnp.minimum(m2, t1)
      n2 = jnp.maximum(m2, t1)
      n3 = jnp.maximum(m3, t2)
      j1 = jnp.where(c1, iv, i1)
      j2 = jnp.where(c1, i1, jnp.where(c2, iv, i2))
      j3 = jnp.where(c2, i2, jnp.where(c3, iv, i3))
      return n1, n2, n3, j1, j2, j3, iv + _L

    m1, m2, m3, i1, i2, i3, _ = lax.fori_loop(
        0, _NV, body,
        (neg_inf, neg_inf, neg_inf, zeros_i, zeros_i, zeros_i, iota))

    vacc = neg_inf
    iacc = zeros_i
    for k in range(_K):
      s = _bcast_max(m1)
      cand = _bcast_min_i32(jnp.where(m1 == s, i1, big))
      vacc = jnp.where(iota == k, s, vacc)
      iacc = jnp.where(iota == k, cand, iacc)
      hit = (m1 == s) & (i1 == cand)
      m1 = jnp.where(hit, m2, m1)
      i1 = jnp.where(hit, i2, i1)
      m2 = jnp.where(hit, m3, m2)
      i2 = jnp.where(hit, i3, i2)
      m3 = jnp.where(hit, neg_inf, m3)
    vout[r, :] = vacc
    iout[r, :] = iacc
    return 0

  lax.fori_loop(0, _RPW, row_body, 0)

  pltpu.sync_copy(vout, vals_hbm.at[pl.ds(base, _RPW)])
  pltpu.sync_copy(iout, idxs_hbm.at[pl.ds(base, _RPW)])


_topk_sc = functools.partial(
    pl.kernel,
    out_type=(jax.ShapeDtypeStruct((_R, _OPAD), jnp.float32),
              jax.ShapeDtypeStruct((_R, _OPAD), jnp.int32)),
    mesh=plsc.VectorSubcoreMesh(
        core_axis_name="c", subcore_axis_name="s",
        num_cores=_NC, num_subcores=_NS),
    compiler_params=pltpu.CompilerParams(needs_layout_passes=False),
    scratch_types=[
        pltpu.VMEM((_RPW, _C), jnp.float32),
        pltpu.VMEM((_RPW, _OPAD), jnp.float32),
        pltpu.VMEM((_RPW, _OPAD), jnp.int32),
    ],
)(_topk_body)


@jax.jit
def kernel(x):
  vals, idxs = _topk_sc(x)
  return vals[:, :_K], idxs[:, :_K]


# speedup vs baseline: 1.0452x; 1.0345x over previous
import functools

import jax
import jax.numpy as jnp
from jax import lax
from jax.experimental import pallas as pl
from jax.experimental.pallas import tpu as pltpu
from jax.experimental.pallas import tpu_sc as plsc

_NC = 2
_NS = 16
_NW = _NC * _NS
_L = 16

_R = 128
_C = 8192
_K = 3
_RPW = _R // _NW
_NV = _C // _L
_OPAD = 16


def _bcast_max(x):
  return plsc.cummax(jnp.flip(plsc.cummax(x)))


def _bcast_min_i32(x):
  return -_bcast_max(-x)


def _topk_body(x_hbm, out_hbm, xin, vout):
  w = lax.axis_index("s") * _NC + lax.axis_index("c")
  base = w * _RPW
  pltpu.sync_copy(x_hbm.at[pl.ds(base, _RPW)], xin)

  neg_inf = jnp.full((_L,), -jnp.inf, jnp.float32)
  zeros_i = jnp.zeros((_L,), jnp.int32)
  iota = lax.iota(jnp.int32, _L)
  big = jnp.full((_L,), jnp.int32(2**30))

  def body(t, carry):
    new = []
    iv = carry[-1]
    for r in range(_RPW):
      m1, m2, m3, i1, i2, i3 = carry[r]
      v = xin[r, pl.ds(t * _L, _L)]
      c1 = v > m1
      c2 = v > m2
      c3 = v > m3
      t1 = jnp.minimum(m1, v)
      n1 = jnp.maximum(m1, v)
      t2 = jnp.minimum(m2, t1)
      n2 = jnp.maximum(m2, t1)
      n3 = jnp.maximum(m3, t2)
      j1 = jnp.where(c1, iv, i1)
      j2 = jnp.where(c1, i1, jnp.where(c2, iv, i2))
      j3 = jnp.where(c2, i2, jnp.where(c3, iv, i3))
      new.append((n1, n2, n3, j1, j2, j3))
    return (*new, iv + _L)

  init = tuple((neg_inf, neg_inf, neg_inf, zeros_i, zeros_i, zeros_i)
               for _ in range(_RPW))
  carry = lax.fori_loop(0, _NV, body, (*init, iota))

  for r in range(_RPW):
    m1, m2, m3, i1, i2, i3 = carry[r]
    vacc = neg_inf
    iacc = zeros_i
    for k in range(_K):
      s = _bcast_max(m1)
      cand = _bcast_min_i32(jnp.where(m1 == s, i1, big))
      vacc = jnp.where(iota == k, s, vacc)
      iacc = jnp.where(iota == k, cand, iacc)
      hit = (m1 == s) & (i1 == cand)
      m1 = jnp.where(hit, m2, m1)
      i1 = jnp.where(hit, i2, i1)
      m2 = jnp.where(hit, m3, m2)
      i2 = jnp.where(hit, i3, i2)
      m3 = jnp.where(hit, neg_inf, m3)
    vout[0, r, :] = vacc
    vout[1, r, :] = plsc.bitcast(iacc, jnp.float32)

  pltpu.sync_copy(vout.at[0], out_hbm.at[0, pl.ds(base, _RPW)])
  pltpu.sync_copy(vout.at[1], out_hbm.at[1, pl.ds(base, _RPW)])


_topk_sc = functools.partial(
    pl.kernel,
    out_type=jax.ShapeDtypeStruct((2, _R, _OPAD), jnp.float32),
    mesh=plsc.VectorSubcoreMesh(
        core_axis_name="c", subcore_axis_name="s",
        num_cores=_NC, num_subcores=_NS),
    compiler_params=pltpu.CompilerParams(needs_layout_passes=False),
    scratch_types=[
        pltpu.VMEM((_RPW, _C), jnp.float32),
        pltpu.VMEM((2, _RPW, _OPAD), jnp.float32),
    ],
)(_topk_body)


@jax.jit
def kernel(x):
  out = _topk_sc(x)
  sliced = out[:, :, :_K]
  vals = sliced[0]
  idxs = lax.bitcast_convert_type(sliced[1], jnp.int32)
  return vals, idxs


# speedup vs baseline: 1.0465x; 1.0013x over previous
import functools

import jax
import jax.numpy as jnp
from jax import lax
from jax.experimental import pallas as pl
from jax.experimental.pallas import tpu as pltpu
from jax.experimental.pallas import tpu_sc as plsc

_NC = 2
_NS = 16
_NW = _NC * _NS
_L = 16

_R = 128
_C = 8192
_K = 3
_RPW = _R // _NW
_NV = _C // _L
_OPAD = 16


def _bcast_max(x):
  return plsc.cummax(jnp.flip(plsc.cummax(x)))


def _bcast_min_i32(x):
  return -_bcast_max(-x)


def _topk_body(x_hbm, out_hbm, xin, vout):
  w = lax.axis_index("s") * _NC + lax.axis_index("c")
  base = w * _RPW
  pltpu.sync_copy(x_hbm.at[pl.ds(base, _RPW)], xin)

  neg_inf = jnp.full((_L,), -jnp.inf, jnp.float32)
  zeros_i = jnp.zeros((_L,), jnp.int32)
  iota = lax.iota(jnp.int32, _L)
  big = jnp.full((_L,), jnp.int32(2**30))

  def body(t, carry):
    new = []
    iv = carry[-1]
    for r in range(_RPW):
      m1, m2, m3, i1, i2, i3 = carry[r]
      v = xin[r, pl.ds(t * _L, _L)]
      c1 = v > m1
      c2 = v > m2
      c3 = v > m3
      t1 = jnp.minimum(m1, v)
      n1 = jnp.maximum(m1, v)
      t2 = jnp.minimum(m2, t1)
      n2 = jnp.maximum(m2, t1)
      n3 = jnp.maximum(m3, t2)
      j1 = jnp.where(c1, iv, i1)
      j2 = jnp.where(c1, i1, jnp.where(c2, iv, i2))
      j3 = jnp.where(c2, i2, jnp.where(c3, iv, i3))
      new.append((n1, n2, n3, j1, j2, j3))
    return (*new, iv + _L)

  init = tuple((neg_inf, neg_inf, neg_inf, zeros_i, zeros_i, zeros_i)
               for _ in range(_RPW))
  carry = lax.fori_loop(0, _NV, body, (*init, iota))

  for r in range(_RPW):
    m1, m2, m3, i1, i2, i3 = carry[r]
    vacc = neg_inf
    iacc = zeros_i
    for k in range(_K):
      s = _bcast_max(m1)
      cand = _bcast_min_i32(jnp.where(m1 == s, i1, big))
      vacc = jnp.where(iota == k, s, vacc)
      iacc = jnp.where(iota == k, cand, iacc)
      hit = (m1 == s) & (i1 == cand)
      m1 = jnp.where(hit, m2, m1)
      i1 = jnp.where(hit, i2, i1)
      m2 = jnp.where(hit, m3, m2)
      i2 = jnp.where(hit, i3, i2)
      m3 = jnp.where(hit, neg_inf, m3)
    vout[0, r, :] = vacc
    vout[1, r, :] = plsc.bitcast(iacc, jnp.float32)

  pltpu.sync_copy(vout.at[0], out_hbm.at[0, pl.ds(base, _RPW)])
  pltpu.sync_copy(vout.at[1], out_hbm.at[1, pl.ds(base, _RPW)])


_topk_sc = functools.partial(
    pl.kernel,
    out_type=jax.ShapeDtypeStruct((2, _R, _OPAD), jnp.float32),
    mesh=plsc.VectorSubcoreMesh(
        core_axis_name="c", subcore_axis_name="s",
        num_cores=_NC, num_subcores=_NS),
    compiler_params=pltpu.CompilerParams(
        needs_layout_passes=False, skip_device_barrier=True),
    scratch_types=[
        pltpu.VMEM((_RPW, _C), jnp.float32),
        pltpu.VMEM((2, _RPW, _OPAD), jnp.float32),
    ],
)(_topk_body)


@jax.jit
def kernel(x):
  out = _topk_sc(x)
  sliced = out[:, :, :_K]
  vals = sliced[0]
  idxs = lax.bitcast_convert_type(sliced[1], jnp.int32)
  return vals, idxs


# speedup vs baseline: 1.0564x; 1.0095x over previous
import functools

import jax
import jax.numpy as jnp
from jax import lax
from jax.experimental import pallas as pl
from jax.experimental.pallas import tpu as pltpu
from jax.experimental.pallas import tpu_sc as plsc

_NC = 2
_NS = 16
_NW = _NC * _NS
_L = 16

_R = 128
_C = 8192
_K = 3
_RPW = _R // _NW
_NV = _C // _L
_OPAD = 16


def _bcast_max(x):
  return plsc.cummax(jnp.flip(plsc.cummax(x)))


def _bcast_min_i32(x):
  return -_bcast_max(-x)


def _topk_body(x_hbm, out_hbm, xin, vout):
  w = lax.axis_index("s") * _NC + lax.axis_index("c")
  base = w * _RPW
  pltpu.sync_copy(x_hbm.at[pl.ds(base, _RPW)], xin)

  neg_inf = jnp.full((_L,), -jnp.inf, jnp.float32)
  zeros_i = jnp.zeros((_L,), jnp.int32)
  iota = lax.iota(jnp.int32, _L)
  big = jnp.full((_L,), jnp.int32(2**30))

  def body(t, carry):
    new = []
    iv = carry[-1]
    for r in range(_RPW):
      m1, m2, m3, i1, i2, i3 = carry[r]
      v = xin[r, pl.ds(t * _L, _L)]
      c1 = v > m1
      c2 = v > m2
      c3 = v > m3
      t1 = jnp.minimum(m1, v)
      n1 = jnp.maximum(m1, v)
      t2 = jnp.minimum(m2, t1)
      n2 = jnp.maximum(m2, t1)
      n3 = jnp.maximum(m3, t2)
      j1 = jnp.where(c1, iv, i1)
      j2 = jnp.where(c1, i1, jnp.where(c2, iv, i2))
      j3 = jnp.where(c2, i2, jnp.where(c3, iv, i3))
      new.append((n1, n2, n3, j1, j2, j3))
    return (*new, iv + _L)

  init = tuple((neg_inf, neg_inf, neg_inf, zeros_i, zeros_i, zeros_i)
               for _ in range(_RPW))
  carry = lax.fori_loop(0, _NV, body, (*init, iota), unroll=2)

  for r in range(_RPW):
    m1, m2, m3, i1, i2, i3 = carry[r]
    vacc = neg_inf
    iacc = zeros_i
    for k in range(_K):
      s = _bcast_max(m1)
      cand = _bcast_min_i32(jnp.where(m1 == s, i1, big))
      vacc = jnp.where(iota == k, s, vacc)
      iacc = jnp.where(iota == k, cand, iacc)
      hit = (m1 == s) & (i1 == cand)
      m1 = jnp.where(hit, m2, m1)
      i1 = jnp.where(hit, i2, i1)
      m2 = jnp.where(hit, m3, m2)
      i2 = jnp.where(hit, i3, i2)
      m3 = jnp.where(hit, neg_inf, m3)
    vout[0, r, :] = vacc
    vout[1, r, :] = plsc.bitcast(iacc, jnp.float32)

  pltpu.sync_copy(vout.at[0], out_hbm.at[0, pl.ds(base, _RPW)])
  pltpu.sync_copy(vout.at[1], out_hbm.at[1, pl.ds(base, _RPW)])


_topk_sc = functools.partial(
    pl.kernel,
    out_type=jax.ShapeDtypeStruct((2, _R, _OPAD), jnp.float32),
    mesh=plsc.VectorSubcoreMesh(
        core_axis_name="c", subcore_axis_name="s",
        num_cores=_NC, num_subcores=_NS),
    compiler_params=pltpu.CompilerParams(needs_layout_passes=False),
    scratch_types=[
        pltpu.VMEM((_RPW, _C), jnp.float32),
        pltpu.VMEM((2, _RPW, _OPAD), jnp.float32),
    ],
)(_topk_body)


@jax.jit
def kernel(x):
  out = _topk_sc(x)
  sliced = out[:, :, :_K]
  vals = sliced[0]
  idxs = lax.bitcast_convert_type(sliced[1], jnp.int32)
  return vals, idxs
